# Initial kernel scaffold; baseline (speedup 1.0000x reference)
import jax
import jax.numpy as jnp
from jax.experimental import pallas as pl


def kernel(x, g, Wq, bq, Wk, bk, Wv, bv, Wo, bo, ln1_g, ln1_b, Wf1, bf1, Wf2, bf2, ln2_g, ln2_b):
    raise NotImplementedError("write your pallas kernel here")


# speedup vs baseline: 9.1218x; 9.1218x over previous
import functools

import jax
import jax.numpy as jnp
import numpy as np
from jax import lax
from jax.experimental import pallas as pl
from jax.experimental.pallas import tpu as pltpu
from jax.experimental.pallas import tpu_sc as plsc

N = 10000
E = 320000
D = 128
H = 8
DH = 16

NCORES = 2
NSUB = 16
NW = NCORES * NSUB

GSZ = 16
EPAD = 331776
EPT = EPAD // NSUB
ROWS = 10240
HALF = ROWS // NCORES
TRASH = HALF
RPT = HALF // NSUB


def _qkv_body(x_ref, wq_ref, bq_ref, wk_ref, bk_ref, wv_ref, bv_ref, s_ref,
              scores_ref, v_ref):
    x = x_ref[...]
    q = jnp.dot(x, wq_ref[...], preferred_element_type=jnp.float32) + bq_ref[...]
    k = jnp.dot(x, wk_ref[...], preferred_element_type=jnp.float32) + bk_ref[...]
    v = jnp.dot(x, wv_ref[...], preferred_element_type=jnp.float32) + bv_ref[...]
    qk = q * k
    scores_ref[...] = jnp.dot(qk, s_ref[...],
                              preferred_element_type=jnp.float32) * 0.25
    v_ref[...] = v


def _qkv_call(x, wq, bq, wk, bk, wv, bv, sel):
    blk = 1000
    grid = (N // blk,)
    full = lambda shape: pl.BlockSpec(shape, lambda i: (0,) * len(shape))
    return pl.pallas_call(
        _qkv_body,
        grid=grid,
        in_specs=[
            pl.BlockSpec((blk, D), lambda i: (i, 0)),
            full((D, D)), full((1, D)),
            full((D, D)), full((1, D)),
            full((D, D)), full((1, D)),
            full((D, D)),
        ],
        out_specs=[
            pl.BlockSpec((blk, D), lambda i: (i, 0)),
            pl.BlockSpec((blk, D), lambda i: (i, 0)),
        ],
        out_shape=[
            jax.ShapeDtypeStruct((N, D), jnp.float32),
            jax.ShapeDtypeStruct((N, D), jnp.float32),
        ],
    )(x, wq, bq, wk, bk, wv, bv, sel)


def _edge_body(scores_hbm, v_hbm, src_hbm, dst_hbm, dstc_hbm,
               num_out, den_out,
               s1g, s2g, vg, pbuf16, sidx16, dcidx16, didx16, didx2b,
               zbuf, num_sh, den_sh, sem):
    cid = lax.axis_index("c")
    sid = lax.axis_index("s")
    ebase = sid * EPT
    rbase = sid * RPT

    zero16 = jnp.zeros((16,), jnp.float32)

    def zrow(r, _):
        for j in range(D // 16):
            zbuf[r, pl.ds(j * 16, 16)] = zero16
        return 0

    lax.fori_loop(0, 128, zrow, 0)

    for r in range(2):
        pltpu.sync_copy(zbuf, num_sh.at[pl.ds(rbase + r * 128, 128)])
        pltpu.sync_copy(zbuf, den_sh.at[pl.ds(rbase + r * 128, 128)])
    pltpu.sync_copy(zbuf.at[pl.ds(0, RPT - 256)],
                    num_sh.at[pl.ds(rbase + 256, RPT - 256)])
    pltpu.sync_copy(zbuf.at[pl.ds(0, RPT - 256)],
                    den_sh.at[pl.ds(rbase + 256, RPT - 256)])
    for i in range(GSZ):
        for j in range(D // 16):
            pbuf16[i, pl.ds(j * 16, 16)] = zero16
    plsc.subcore_barrier()

    rlo = cid * HALF

    def group(g, _):
        gbase = ebase + g * GSZ
        pltpu.sync_copy(src_hbm.at[pl.ds(gbase, GSZ)], sidx16)
        pltpu.sync_copy(dst_hbm.at[pl.ds(gbase, GSZ)], didx16)
        pltpu.sync_copy(dstc_hbm.at[pl.ds(gbase, GSZ)], dcidx16)
        pltpu.async_copy(v_hbm.at[sidx16], vg, sem).wait()
        pltpu.async_copy(scores_hbm.at[sidx16], s1g, sem).wait()
        pltpu.async_copy(scores_hbm.at[dcidx16], s2g, sem).wait()

        dv = didx16[:] - rlo
        ok = (dv >= 0) & (dv < HALF)
        didx2b[:] = jnp.where(ok, dv, TRASH)

        for i in range(GSZ):
            a = s1g[i, pl.ds(0, 16)] + s2g[i, pl.ds(0, 16)]
            a = jnp.where(a >= 0.0, a, a * 0.2)
            p = jnp.exp(a)
            pbuf16[i, pl.ds(0, 16)] = p
            for h in range(H):
                vg[i, pl.ds(h * 16, 16)] = vg[i, pl.ds(h * 16, 16)] * p[h]

        pltpu.sync_copy(pbuf16, den_sh.at[didx2b], add=True)
        pltpu.sync_copy(vg, num_sh.at[didx2b], add=True)
        return 0

    lax.fori_loop(0, EPT // GSZ, group, 0)
    plsc.subcore_barrier()

    pltpu.sync_copy(num_sh.at[pl.ds(rbase, RPT)],
                    num_out.at[pl.ds(cid * HALF + rbase, RPT)])
    pltpu.sync_copy(den_sh.at[pl.ds(rbase, RPT)],
                    den_out.at[pl.ds(cid * HALF + rbase, RPT)])


def _edge_call(scores, v, src, dst, dstc):
    mesh = plsc.VectorSubcoreMesh(core_axis_name="c", subcore_axis_name="s")
    return pl.kernel(
        _edge_body,
        out_type=[
            jax.ShapeDtypeStruct((ROWS, D), jnp.float32),
            jax.ShapeDtypeStruct((ROWS, D), jnp.float32),
        ],
        mesh=mesh,
        scratch_types=[
            pltpu.VMEM((GSZ, D), jnp.float32),
            pltpu.VMEM((GSZ, D), jnp.float32),
            pltpu.VMEM((GSZ, D), jnp.float32),
            pltpu.VMEM((GSZ, D), jnp.float32),
            pltpu.VMEM((GSZ,), jnp.int32),
            pltpu.VMEM((GSZ,), jnp.int32),
            pltpu.VMEM((GSZ,), jnp.int32),
            pltpu.VMEM((GSZ,), jnp.int32),
            pltpu.VMEM((128, D), jnp.float32),
            pltpu.VMEM_SHARED((HALF + 8, D), jnp.float32),
            pltpu.VMEM_SHARED((HALF + 8, D), jnp.float32),
            pltpu.SemaphoreType.DMA,
        ],
    )(scores, v, src, dst, dstc)


def _post_body(num_ref, den_ref, x_ref, selt_ref, wo_ref, bo_ref,
               ln1g_ref, ln1b_ref, wf1_ref, bf1_ref, wf2_ref, bf2_ref,
               ln2g_ref, ln2b_ref, out_ref):
    num = num_ref[...]
    den = den_ref[...]
    dexp = jnp.dot(den, selt_ref[...], preferred_element_type=jnp.float32)
    attn = num / (dexp + 1e-16)
    h = jnp.dot(attn, wo_ref[...], preferred_element_type=jnp.float32)
    h = h + bo_ref[...] + x_ref[...]

    mu = jnp.mean(h, axis=-1, keepdims=True)
    var = jnp.mean((h - mu) ** 2, axis=-1, keepdims=True)
    h = (h - mu) * lax.rsqrt(var + 1e-5) * ln1g_ref[...] + ln1b_ref[...]

    f = jnp.dot(h, wf1_ref[...], preferred_element_type=jnp.float32)
    f = jnp.maximum(f + bf1_ref[...], 0.0)
    f = jnp.dot(f, wf2_ref[...], preferred_element_type=jnp.float32)
    h = f + bf2_ref[...] + h

    mu = jnp.mean(h, axis=-1, keepdims=True)
    var = jnp.mean((h - mu) ** 2, axis=-1, keepdims=True)
    out_ref[...] = (h - mu) * lax.rsqrt(var + 1e-5) * ln2g_ref[...] + ln2b_ref[...]


def _post_call(num, den, x, selt, wo, bo, ln1g, ln1b, wf1, bf1, wf2, bf2,
               ln2g, ln2b):
    blk = 1000
    grid = (N // blk,)
    full = lambda shape: pl.BlockSpec(shape, lambda i: (0,) * len(shape))
    return pl.pallas_call(
        _post_body,
        grid=grid,
        in_specs=[
            pl.BlockSpec((blk, D), lambda i: (i, 0)),
            pl.BlockSpec((blk, D), lambda i: (i, 0)),
            pl.BlockSpec((blk, D), lambda i: (i, 0)),
            full((D, D)),
            full((D, D)), full((1, D)),
            full((1, D)), full((1, D)),
            full((D, 2 * D)), full((1, 2 * D)),
            full((2 * D, D)), full((1, D)),
            full((1, D)), full((1, D)),
        ],
        out_specs=pl.BlockSpec((blk, D), lambda i: (i, 0)),
        out_shape=jax.ShapeDtypeStruct((N, D), jnp.float32),
    )(num, den, x, selt, wo, bo, ln1g, ln1b, wf1, bf1, wf2, bf2, ln2g, ln2b)


@jax.jit
def kernel(x, g, Wq, bq, Wk, bk, Wv, bv, Wo, bo, ln1_g, ln1_b, Wf1, bf1,
           Wf2, bf2, ln2_g, ln2_b):
    sel_np = np.zeros((D, D), np.float32)
    for hh in range(H):
        sel_np[hh * DH:(hh + 1) * DH, hh] = 1.0
    sel = jnp.asarray(sel_np)

    scores, v = _qkv_call(x, Wq, bq.reshape(1, D), Wk, bk.reshape(1, D),
                          Wv, bv.reshape(1, D), sel)

    src0 = g[0]
    dst0 = jnp.where(src0 != g[1], g[1], N)
    loop = jnp.arange(N, dtype=jnp.int32)
    npad = EPAD - (E + N)
    src = jnp.concatenate([src0, loop, jnp.zeros((npad,), jnp.int32)])
    dst = jnp.concatenate([dst0, loop, jnp.full((npad,), N, jnp.int32)])
    dstc = jnp.minimum(dst, N - 1)

    num, den = _edge_call(scores, v, src, dst, dstc)

    selt = jnp.asarray(sel_np.T)

    return _post_call(num, den, x, selt, Wo, bo.reshape(1, D),
                      ln1_g.reshape(1, D), ln1_b.reshape(1, D),
                      Wf1, bf1.reshape(1, 2 * D), Wf2, bf2.reshape(1, D),
                      ln2_g.reshape(1, D), ln2_b.reshape(1, D))
